# baseline (speedup 1.0000x reference)
import functools

import jax
import jax.numpy as jnp
from jax import lax
from jax.experimental import pallas as pl
from jax.experimental.pallas import tpu as pltpu
from jax.experimental.pallas import tpu_sc as plsc

B, N, M = 16, 1024, 1024
GAMMA = 0.5


def _prep_body(g1_ref, c_ref):
    g1 = g1_ref[0]
    riota = lax.broadcasted_iota(jnp.int32, (N + 1, M), 0)
    skip = (riota == g1).astype(jnp.int32)
    p = skip
    k = 1
    while k < M:
        p = p + jnp.concatenate(
            [jnp.zeros((N + 1, k), jnp.int32), p[:, : M - k]], axis=1)
        k *= 2
    excl = p - skip
    rowtot = p[:, M - 1: M].astype(jnp.float32)
    rio = lax.broadcasted_iota(jnp.int32, (N + 1, N + 1), 0)
    cio = lax.broadcasted_iota(jnp.int32, (N + 1, N + 1), 1)
    ltri = (rio > cio).astype(jnp.float32)
    rowbase = lax.dot_general(ltri, rowtot, (((1,), (0,)), ((), ())),
                              preferred_element_type=jnp.float32)
    c_ref[0] = rowbase.astype(jnp.int32) + excl


def _prep(g1_3d):
    return pl.pallas_call(
        _prep_body,
        grid=(B,),
        in_specs=[pl.BlockSpec((1, 1, M), lambda i: (i, 0, 0))],
        out_specs=pl.BlockSpec((1, N + 1, M), lambda i: (i, 0, 0)),
        out_shape=jax.ShapeDtypeStruct((B, N + 1, M), jnp.int32),
    )(g1_3d)


def _sc_cross_body(scores_hbm, g1_hbm, c_hbm, out_hbm,
                   sbuf, cbuf, g1buf, accbuf):
    cid = lax.axis_index("c")
    sid = lax.axis_index("s")
    wid = cid * 16 + sid
    bb = wid // 2
    half = wid % 2
    i0 = half * 512

    def zbody(k, _):
        accbuf[pl.ds(k * 16, 16)] = jnp.zeros((16,), jnp.float32)
        return 0
    lax.fori_loop(0, 128, zbody, 0)

    pltpu.sync_copy(g1_hbm.at[bb], g1buf)

    def do_slice(r, i, s):
        g1s = g1buf[0, pl.ds(s * 16, 16)]
        vals = sbuf[r, pl.ds(s * 16, 16)]
        cvec = cbuf[r, pl.ds(s * 16, 16)]
        mask = g1s == i
        idx = lax.iota(jnp.int32, 16) + (s * 16 + M) - cvec
        plsc.addupdate_scatter(accbuf, [idx], vals,
                               mask=jnp.logical_not(mask))

    def chunk_body(ch, _):
        rstart = i0 + ch * 8
        pltpu.sync_copy(scores_hbm.at[bb, pl.ds(rstart, 8), :], sbuf)
        pltpu.sync_copy(c_hbm.at[bb, pl.ds(rstart, 8), :], cbuf)

        def row_body(r, _):
            i = rstart + r

            def s_body(t, _):
                for su in range(4):
                    do_slice(r, i, t * 4 + su)
                return 0
            lax.fori_loop(0, 16, s_body, 0)
            return 0
        lax.fori_loop(0, 8, row_body, 0)
        return 0
    lax.fori_loop(0, 64, chunk_body, 0)

    @pl.when(half == 1)
    def _():
        pltpu.sync_copy(scores_hbm.at[bb, pl.ds(N, 1), :],
                        sbuf.at[pl.ds(0, 1)])
        pltpu.sync_copy(c_hbm.at[bb, pl.ds(N, 1), :], cbuf.at[pl.ds(0, 1)])

        def s_body(t, _):
            for su in range(4):
                do_slice(0, N, t * 4 + su)
            return 0
        lax.fori_loop(0, 16, s_body, 0)

    pltpu.sync_copy(accbuf, out_hbm.at[wid, 0])


@functools.cache
def _build_sc_cross():
    @functools.partial(
        pl.kernel,
        mesh=plsc.VectorSubcoreMesh(core_axis_name="c", subcore_axis_name="s"),
        compiler_params=pltpu.CompilerParams(needs_layout_passes=False),
        out_type=jax.ShapeDtypeStruct((32, 8, 2 * M), jnp.float32),
        scratch_types=[
            pltpu.VMEM((8, M + 1), jnp.float32),
            pltpu.VMEM((8, M), jnp.int32),
            pltpu.VMEM((1, M), jnp.int32),
            pltpu.VMEM((2 * M,), jnp.float32),
        ],
    )
    def sc_cross(scores_hbm, g1_hbm, c_hbm, out_hbm,
                 sbuf, cbuf, g1buf, accbuf):
        _sc_cross_body(scores_hbm, g1_hbm, c_hbm, out_hbm,
                       sbuf, cbuf, g1buf, accbuf)
    return sc_cross


def _sc_cross(scores, g1, cfull):
    return _build_sc_cross()(scores, g1.reshape(B, 1, M), cfull)[:, 0, :]


def _dense_body(s_ref, g0_ref, g1_ref, part_ref, pos2_ref):
    S = s_ref[0]
    g0 = g0_ref[0]
    g1 = g1_ref[0]
    Sm = S[: N, : M]
    lastcol = S[: N, M: M + 1]
    lastrow = S[N: N + 1, : M]

    colidx = lax.broadcasted_iota(jnp.int32, (N, M), 1)
    rowidx = lax.broadcasted_iota(jnp.int32, (N, M), 0)
    NEG = jnp.float32(-jnp.inf)
    BIGI = jnp.int32(1 << 30)

    ident = (colidx == rowidx).astype(jnp.float32)
    g0col = lax.dot_general(ident, g0.astype(jnp.float32),
                            (((1,), (1,)), ((), ())),
                            preferred_element_type=jnp.float32)
    g1col = lax.dot_general(ident, g1.astype(jnp.float32),
                            (((1,), (1,)), ((), ())),
                            preferred_element_type=jnp.float32)
    g0c = g0col.astype(jnp.int32)

    rowsum = jnp.sum(Sm, axis=1, keepdims=True) + lastcol
    rowsumsq = jnp.sum(Sm * Sm, axis=1, keepdims=True) + lastcol * lastcol
    rmax_m = jnp.max(Sm, axis=1, keepdims=True)
    idx_in_m = jnp.min(jnp.where(Sm == rmax_m, colidx, BIGI),
                       axis=1, keepdims=True)
    max1 = jnp.maximum(rmax_m, lastcol)
    idx1 = jnp.where(rmax_m >= lastcol, idx_in_m, M)
    m2m = jnp.max(jnp.where(colidx == idx_in_m, NEG, Sm),
                  axis=1, keepdims=True)
    t2 = jnp.where(idx1 == M, rmax_m, jnp.maximum(m2m, lastcol))
    pos1 = (jnp.sum(jnp.where(colidx == g0c, Sm, 0.0), axis=1, keepdims=True)
            + jnp.where(g0c == M, lastcol, 0.0))
    hn1 = jnp.where(idx1 == g0c, t2, max1)
    h1 = jnp.sum(jnp.maximum(pos1 - hn1 + GAMMA, 0.0))
    sumn = rowsum - pos1
    sum_m1 = jnp.sum(sumn - M * pos1)
    sumsq_m1 = jnp.sum((rowsumsq - pos1 * pos1)
                       - 2.0 * pos1 * sumn + M * pos1 * pos1)

    colsum = jnp.sum(Sm, axis=0, keepdims=True) + lastrow
    colsumsq = jnp.sum(Sm * Sm, axis=0, keepdims=True) + lastrow * lastrow
    sum_sb = jnp.sum(colsum)
    sumsq_sb = jnp.sum(colsumsq)
    cmax_m = jnp.max(Sm, axis=0, keepdims=True)
    cidx_in_m = jnp.min(jnp.where(Sm == cmax_m, rowidx, BIGI),
                        axis=0, keepdims=True)
    u1 = jnp.maximum(cmax_m, lastrow)
    cidx1 = jnp.where(cmax_m >= lastrow, cidx_in_m, N)
    c2m = jnp.max(jnp.where(rowidx == cidx_in_m, NEG, Sm),
                  axis=0, keepdims=True)
    u2 = jnp.where(cidx1 == N, cmax_m, jnp.maximum(c2m, lastrow))
    P = (jnp.sum(jnp.where(rowidx == g1, Sm, 0.0), axis=0, keepdims=True)
         + jnp.where(g1 == N, lastrow, 0.0))
    hn2 = jnp.where(cidx1 == g1, u2, u1)

    g1row_f = g1.astype(jnp.float32)
    lt = (g1row_f < g1col).astype(jnp.float32)
    eq = (g1row_f == g1col).astype(jnp.float32)
    jplt = (colidx < rowidx).astype(jnp.float32)
    rank = jnp.sum(lt + eq * jplt, axis=1, keepdims=True)
    oh = (rank == colidx.astype(jnp.float32)).astype(jnp.float32)
    pos2 = lax.dot_general(P, oh, (((1,), (0,)), ((), ())),
                           preferred_element_type=jnp.float32)
    h2 = jnp.sum(jnp.maximum(pos2 - hn2 + GAMMA, 0.0))
    sp = jnp.sum(P)
    sp2 = jnp.sum(P * P)
    sum_m2 = (sum_sb - sp) - N * sp
    sumsq_m2_wo = (sumsq_sb - sp2) + N * sp2

    lane = lax.broadcasted_iota(jnp.int32, (1, 128), 1)
    vals = (jnp.where(lane == 0, sum_m1, 0.0)
            + jnp.where(lane == 1, sumsq_m1, 0.0)
            + jnp.where(lane == 2, sum_m2, 0.0)
            + jnp.where(lane == 3, sumsq_m2_wo, 0.0)
            + jnp.where(lane == 4, h1, 0.0)
            + jnp.where(lane == 5, h2, 0.0))
    part_ref[0] = vals
    pos2_ref[0] = pos2


def _dense(scores, g0_3d, g1_3d):
    return pl.pallas_call(
        _dense_body,
        grid=(B,),
        in_specs=[
            pl.BlockSpec((1, N + 1, M + 1), lambda i: (i, 0, 0)),
            pl.BlockSpec((1, 1, N), lambda i: (i, 0, 0)),
            pl.BlockSpec((1, 1, M), lambda i: (i, 0, 0)),
        ],
        out_specs=[
            pl.BlockSpec((1, 1, 128), lambda i: (i, 0, 0)),
            pl.BlockSpec((1, 1, M), lambda i: (i, 0, 0)),
        ],
        out_shape=[
            jax.ShapeDtypeStruct((B, 1, 128), jnp.float32),
            jax.ShapeDtypeStruct((B, 1, M), jnp.float32),
        ],
    )(scores, g0_3d, g1_3d)


def _final_body(part_ref, pos2_ref, acc_ref, out_ref):
    acc = acc_ref[...]
    s = acc[:, 0, :] + acc[:, 1, :]
    fold = s[:, : M] + s[:, M:]
    pos2 = pos2_ref[:, 0, :]
    cross = jnp.sum(pos2 * fold)
    part = part_ref[:, 0, :]
    sum_m1 = jnp.sum(part[:, 0:1])
    sumsq_m1 = jnp.sum(part[:, 1:2])
    sum_m2 = jnp.sum(part[:, 2:3])
    sumsq_m2 = jnp.sum(part[:, 3:4]) - 2.0 * cross
    h = jnp.sum(part[:, 4:5]) + jnp.sum(part[:, 5:6])
    ntot = jnp.float32(2 * B * N * M)
    mu = (sum_m1 + sum_m2) / ntot
    var = (sumsq_m1 + sumsq_m2 - ntot * mu * mu) / (ntot - 1.0)
    hard = h / jnp.float32(2 * B * N)
    out_ref[...] = (hard + jnp.exp(mu) + jnp.log(var + 1.0)).reshape(1, 1)


def _final(partials, pos2_all, acc):
    return pl.pallas_call(
        _final_body,
        out_shape=jax.ShapeDtypeStruct((1, 1), jnp.float32),
    )(partials, pos2_all, acc.reshape(B, 2, 2 * M))


def kernel(gt_matches0, gt_matches1, scores):
    g0 = jnp.where(gt_matches0 == -1, M, gt_matches0).astype(jnp.int32)
    g1 = jnp.where(gt_matches1 == -1, N, gt_matches1).astype(jnp.int32)
    g0_3d = g0.reshape(B, 1, N)
    g1_3d = g1.reshape(B, 1, M)
    cfull = _prep(g1_3d)
    acc = _sc_cross(scores, g1, cfull)
    partials, pos2_all = _dense(scores, g0_3d, g1_3d)
    out = _final(partials, pos2_all, acc)
    return out[0, 0]
